# baseline (speedup 1.0000x reference)
import functools
import math

import jax
import jax.numpy as jnp
from jax import lax
from jax.experimental import pallas as pl
from jax.experimental.pallas import tpu as pltpu
from jax.experimental.pallas import tpu_sc as plsc

VOCAB = 1000000
EMB_DIM = 64
SCALE = math.sqrt(EMB_DIM)

NC = 2
NS = 16
NW = NC * NS
LANES = 16

BW = 128
GRP = 8
NBUF = 4
PITCH = BW + 1

N_COL = (VOCAB + BW - 1) // BW
RING_COLS = (N_COL - 1) // NW
LAST_COL = N_COL - 1
T_PITCH = 2 * EMB_DIM + 3


def _make_transpose(mesh):
    @functools.partial(
        pl.kernel,
        out_type=jax.ShapeDtypeStruct((VOCAB // 2, 2 * EMB_DIM), jnp.float32),
        mesh=mesh,
        scratch_types=[
            [pltpu.VMEM((EMB_DIM, BW), jnp.float32) for _ in range(NBUF)],
            [pltpu.VMEM((EMB_DIM, T_PITCH), jnp.float32) for _ in range(NBUF)],
            [pltpu.SemaphoreType.DMA for _ in range(NBUF)],
            [pltpu.SemaphoreType.DMA for _ in range(NBUF)],
        ],
        compiler_params=pltpu.CompilerParams(
            use_tc_tiling_on_sc=True, needs_layout_passes=False
        ),
    )
    def table_transpose(tab_hbm, last_hbm, out_hbm, stg, obuf, sg, sw):
        wid = lax.axis_index("s") * NC + lax.axis_index("c")
        c0 = wid * RING_COLS

        e_iota = lax.iota(jnp.int32, LANES)

        def fire_read(c, b):
            pltpu.async_copy(tab_hbm.at[:, pl.ds(c * BW, BW)], stg[b], sg[b])

        def wait_read(b):
            pltpu.make_async_copy(
                tab_hbm.at[:, pl.ds(0, BW)], stg[b], sg[b]
            ).wait()

        def fire_write(c, b):
            pltpu.async_copy(
                obuf[b].at[:, pl.ds(0, 2 * EMB_DIM)],
                out_hbm.at[pl.ds(c * (BW // 2), EMB_DIM)],
                sw[b],
            )

        def wait_write(b):
            pltpu.make_async_copy(
                obuf[b].at[:, pl.ds(0, 2 * EMB_DIM)],
                out_hbm.at[pl.ds(0, EMB_DIM)],
                sw[b],
            ).wait()

        d_vecs = [e_iota + (dv * LANES) for dv in range(BW // LANES)]
        q_vecs = [dv // 2 for dv in d_vecs]
        h_vecs = [(dv % 2) * EMB_DIM for dv in d_vecs]

        def transpose(b):
            @plsc.parallel_loop(0, EMB_DIM, unroll=4)
            def e_body(e):
                e_splat = jnp.full((LANES,), 0, jnp.int32) + e
                for dv in range(BW // LANES):
                    vals = stg[b][e, pl.ds(dv * LANES, LANES)] * SCALE
                    plsc.store_scatter(
                        obuf[b], [q_vecs[dv], h_vecs[dv] + e_splat], vals
                    )

        for b in range(NBUF):
            fire_read(c0 + b, b)
        for b in range(NBUF):
            wait_read(b)
            transpose(b)
            fire_write(c0 + b, b)
            fire_read(c0 + b + NBUF, b)

        def main_body(jj, _):
            for b in range(NBUF):
                j = jj * NBUF + b
                wait_read(b)
                wait_write(b)
                transpose(b)
                fire_write(c0 + j, b)
                fire_read(c0 + j + NBUF, b)
            return 0

        lax.fori_loop(1, RING_COLS // NBUF - 1, main_body, 0)

        for b in range(NBUF):
            j = RING_COLS - NBUF + b
            wait_read(b)
            wait_write(b)
            transpose(b)
            fire_write(c0 + j, b)
        for b in range(NBUF):
            wait_write(b)

        @pl.when(wid < N_COL - 1 - RING_COLS * NW)
        def _():
            c = RING_COLS * NW + wid
            pltpu.sync_copy(tab_hbm.at[:, pl.ds(c * BW, BW)], stg[0])
            transpose(0)
            pltpu.sync_copy(
                obuf[0].at[:, pl.ds(0, 2 * EMB_DIM)],
                out_hbm.at[pl.ds(c * (BW // 2), EMB_DIM)],
            )

        @pl.when(wid == NW - 1)
        def _():
            pltpu.sync_copy(last_hbm, stg[0])

            @plsc.parallel_loop(0, EMB_DIM, unroll=4)
            def e_body(e):
                e_splat = jnp.full((LANES,), 0, jnp.int32) + e
                for dv in range(BW // (2 * LANES)):
                    vals = stg[0][e, pl.ds(dv * LANES, LANES)] * SCALE
                    plsc.store_scatter(
                        obuf[0], [q_vecs[dv], h_vecs[dv] + e_splat], vals
                    )

            pltpu.sync_copy(
                obuf[0].at[pl.ds(0, BW // 4), pl.ds(0, 2 * EMB_DIM)],
                out_hbm.at[pl.ds(LAST_COL * (BW // 2), BW // 4)],
            )

    return table_transpose


def _make_gather(s_len, n_win, mesh):
    n_grp = EMB_DIM // GRP

    @functools.partial(
        pl.kernel,
        out_type=jax.ShapeDtypeStruct((s_len, n_grp, n_win, GRP, BW), jnp.float32),
        mesh=mesh,
        scratch_types=[
            pltpu.VMEM((s_len, BW), jnp.int32),
            [pltpu.VMEM((BW, EMB_DIM), jnp.float32) for _ in range(NBUF)],
            [pltpu.VMEM((GRP, GRP, PITCH), jnp.float32) for _ in range(NBUF)],
            [pltpu.SemaphoreType.DMA for _ in range(NBUF)],
            [pltpu.SemaphoreType.DMA for _ in range(NBUF)],
        ],
        compiler_params=pltpu.CompilerParams(
            use_tc_tiling_on_sc=False, needs_layout_passes=False
        ),
    )
    def gather_tr(table_hbm, tok_hbm, out_hbm, idx_v, stg, obuf, sg, sw):
        w = lax.axis_index("s") * NC + lax.axis_index("c")
        pltpu.sync_copy(tok_hbm.at[:, pl.ds(w * BW, BW)], idx_v)

        d_iota = lax.iota(jnp.int32, LANES)

        def fire_gather(s, b):
            pltpu.async_copy(table_hbm.at[idx_v.at[s]], stg[b], sg[b])

        def wait_gather(b):
            pltpu.make_async_copy(table_hbm.at[idx_v.at[0]], stg[b], sg[b]).wait()

        def fire_write(s, b):
            pltpu.async_copy(
                obuf[b].at[:, :, pl.ds(0, BW)], out_hbm.at[s, :, w], sw[b]
            )

        def wait_write(b):
            pltpu.make_async_copy(
                obuf[b].at[:, :, pl.ds(0, BW)], out_hbm.at[0, :, w], sw[b]
            ).wait()

        e_bases = [d_iota + (ev * LANES) for ev in range(EMB_DIM // LANES)]
        g_bases = [eb // GRP for eb in e_bases]
        r_bases = [eb % GRP for eb in e_bases]

        def transpose(b):
            @plsc.parallel_loop(0, BW, unroll=4)
            def d_body(d):
                d_splat = jnp.full((LANES,), 0, jnp.int32) + d
                for ev in range(EMB_DIM // LANES):
                    vals = stg[b][d, pl.ds(ev * LANES, LANES)]
                    plsc.store_scatter(
                        obuf[b], [g_bases[ev], r_bases[ev], d_splat], vals
                    )

        for b in range(NBUF):
            fire_gather(b, b)
        for b in range(NBUF):
            wait_gather(b)
            transpose(b)
            fire_write(b, b)
            fire_gather(b + NBUF, b)

        def main_body(jj, _):
            for b in range(NBUF):
                s = jj * NBUF + b
                wait_gather(b)
                wait_write(b)
                transpose(b)
                fire_write(s, b)
                fire_gather(s + NBUF, b)
            return 0

        lax.fori_loop(1, s_len // NBUF - 1, main_body, 0)

        for b in range(NBUF):
            s = s_len - NBUF + b
            wait_gather(b)
            wait_write(b)
            transpose(b)
            fire_write(s, b)
        for b in range(NBUF):
            wait_write(b)

    return gather_tr


@jax.jit
def kernel(tokens, table):
    b, s = tokens.shape
    assert b % BW == 0 and (b // BW) == NW and s % NBUF == 0
    mesh = plsc.VectorSubcoreMesh(core_axis_name="c", subcore_axis_name="s")
    tail = jnp.concatenate(
        [table[VOCAB - BW // 2 :].T, jnp.zeros((EMB_DIM, BW // 2), jnp.float32)],
        axis=1,
    )
    tab2 = _make_transpose(mesh)(table.T, tail)
    tab_rm = tab2.reshape(VOCAB, EMB_DIM)
    tok_t = tokens.T.astype(jnp.int32)
    out5 = _make_gather(s, b // BW, mesh)(tab_rm, tok_t)
    return out5.transpose(2, 4, 0, 1, 3).reshape(b, s, EMB_DIM)


# speedup vs baseline: 1.0774x; 1.0774x over previous
import functools
import math

import jax
import jax.numpy as jnp
from jax import lax
from jax.experimental import pallas as pl
from jax.experimental.pallas import tpu as pltpu
from jax.experimental.pallas import tpu_sc as plsc

VOCAB = 1000000
EMB_DIM = 64
SCALE = math.sqrt(EMB_DIM)

NC = 2
NS = 16
NW = NC * NS
LANES = 16

BW = 128
GRP = 8
NBUF = 4
PITCH = BW + 1

N_COL = (VOCAB + BW - 1) // BW
RING_COLS = (N_COL - 1) // NW
LAST_COL = N_COL - 1
S_PITCH = BW + 5


def _make_transpose(mesh):
    @functools.partial(
        pl.kernel,
        out_type=jax.ShapeDtypeStruct((VOCAB // 2, 2 * EMB_DIM), jnp.float32),
        mesh=mesh,
        scratch_types=[
            [pltpu.VMEM((EMB_DIM, S_PITCH), jnp.float32) for _ in range(NBUF)],
            [pltpu.VMEM((EMB_DIM, BW), jnp.float32) for _ in range(NBUF)],
            [pltpu.SemaphoreType.DMA for _ in range(NBUF)],
            [pltpu.SemaphoreType.DMA for _ in range(NBUF)],
        ],
        compiler_params=pltpu.CompilerParams(
            use_tc_tiling_on_sc=True, needs_layout_passes=False
        ),
    )
    def table_transpose(tab_hbm, last_hbm, out_hbm, stg, obuf, sg, sw):
        wid = lax.axis_index("s") * NC + lax.axis_index("c")
        c0 = wid * RING_COLS

        e_iota = lax.iota(jnp.int32, LANES)

        def fire_read(c, b):
            pltpu.async_copy(
                tab_hbm.at[:, pl.ds(c * BW, BW)],
                stg[b].at[:, pl.ds(0, BW)],
                sg[b],
            )

        def wait_read(b):
            pltpu.make_async_copy(
                tab_hbm.at[:, pl.ds(0, BW)], stg[b].at[:, pl.ds(0, BW)], sg[b]
            ).wait()

        def fire_write(c, b):
            pltpu.async_copy(
                obuf[b], out_hbm.at[pl.ds(c * (BW // 2), EMB_DIM)], sw[b]
            )

        def wait_write(b):
            pltpu.make_async_copy(
                obuf[b], out_hbm.at[pl.ds(0, EMB_DIM)], sw[b]
            ).wait()

        e_vecs = [e_iota + (ev * LANES) for ev in range(EMB_DIM // LANES)]

        def transpose(b, nd=BW):
            @plsc.parallel_loop(0, nd, unroll=4)
            def d_body(d):
                d_splat = jnp.full((LANES,), 0, jnp.int32) + d
                q = d // 2
                h = (d % 2) * EMB_DIM
                for ev in range(EMB_DIM // LANES):
                    vals = plsc.load_gather(stg[b], [e_vecs[ev], d_splat])
                    obuf[b][q, pl.ds(h + ev * LANES, LANES)] = vals * SCALE

        for b in range(NBUF):
            fire_read(c0 + b, b)
        for b in range(NBUF):
            wait_read(b)
            transpose(b)
            fire_write(c0 + b, b)
            fire_read(c0 + b + NBUF, b)

        def main_body(jj, _):
            for b in range(NBUF):
                j = jj * NBUF + b
                wait_read(b)
                wait_write(b)
                transpose(b)
                fire_write(c0 + j, b)
                fire_read(c0 + j + NBUF, b)
            return 0

        lax.fori_loop(1, RING_COLS // NBUF - 1, main_body, 0)

        for b in range(NBUF):
            j = RING_COLS - NBUF + b
            wait_read(b)
            wait_write(b)
            transpose(b)
            fire_write(c0 + j, b)
        for b in range(NBUF):
            wait_write(b)

        @pl.when(wid < N_COL - 1 - RING_COLS * NW)
        def _():
            c = RING_COLS * NW + wid
            pltpu.sync_copy(
                tab_hbm.at[:, pl.ds(c * BW, BW)], stg[0].at[:, pl.ds(0, BW)]
            )
            transpose(0)
            pltpu.sync_copy(obuf[0], out_hbm.at[pl.ds(c * (BW // 2), EMB_DIM)])

        @pl.when(wid == NW - 1)
        def _():
            pltpu.sync_copy(last_hbm, stg[0].at[:, pl.ds(0, BW)])
            transpose(0, nd=BW // 2)
            pltpu.sync_copy(
                obuf[0].at[pl.ds(0, BW // 4)],
                out_hbm.at[pl.ds(LAST_COL * (BW // 2), BW // 4)],
            )

    return table_transpose


def _make_gather(s_len, n_win, mesh):
    n_grp = EMB_DIM // GRP

    @functools.partial(
        pl.kernel,
        out_type=jax.ShapeDtypeStruct((s_len, n_grp, n_win, GRP, BW), jnp.float32),
        mesh=mesh,
        scratch_types=[
            pltpu.VMEM((s_len, BW), jnp.int32),
            [pltpu.VMEM((BW, EMB_DIM), jnp.float32) for _ in range(NBUF)],
            [pltpu.VMEM((GRP, GRP, PITCH), jnp.float32) for _ in range(NBUF)],
            [pltpu.SemaphoreType.DMA for _ in range(NBUF)],
            [pltpu.SemaphoreType.DMA for _ in range(NBUF)],
        ],
        compiler_params=pltpu.CompilerParams(
            use_tc_tiling_on_sc=False, needs_layout_passes=False
        ),
    )
    def gather_tr(table_hbm, tok_hbm, out_hbm, idx_v, stg, obuf, sg, sw):
        w = lax.axis_index("s") * NC + lax.axis_index("c")
        pltpu.sync_copy(tok_hbm.at[:, pl.ds(w * BW, BW)], idx_v)

        d_iota = lax.iota(jnp.int32, LANES)

        def fire_gather(s, b):
            pltpu.async_copy(table_hbm.at[idx_v.at[s]], stg[b], sg[b])

        def wait_gather(b):
            pltpu.make_async_copy(table_hbm.at[idx_v.at[0]], stg[b], sg[b]).wait()

        def fire_write(s, b):
            pltpu.async_copy(
                obuf[b].at[:, :, pl.ds(0, BW)], out_hbm.at[s, :, w], sw[b]
            )

        def wait_write(b):
            pltpu.make_async_copy(
                obuf[b].at[:, :, pl.ds(0, BW)], out_hbm.at[0, :, w], sw[b]
            ).wait()

        e_bases = [d_iota + (ev * LANES) for ev in range(EMB_DIM // LANES)]
        g_bases = [eb // GRP for eb in e_bases]
        r_bases = [eb % GRP for eb in e_bases]

        def transpose(b):
            @plsc.parallel_loop(0, BW, unroll=4)
            def d_body(d):
                d_splat = jnp.full((LANES,), 0, jnp.int32) + d
                for ev in range(EMB_DIM // LANES):
                    vals = stg[b][d, pl.ds(ev * LANES, LANES)]
                    plsc.store_scatter(
                        obuf[b], [g_bases[ev], r_bases[ev], d_splat], vals
                    )

        for b in range(NBUF):
            fire_gather(b, b)
        for b in range(NBUF):
            wait_gather(b)
            transpose(b)
            fire_write(b, b)
            fire_gather(b + NBUF, b)

        def main_body(jj, _):
            for b in range(NBUF):
                s = jj * NBUF + b
                wait_gather(b)
                wait_write(b)
                transpose(b)
                fire_write(s, b)
                fire_gather(s + NBUF, b)
            return 0

        lax.fori_loop(1, s_len // NBUF - 1, main_body, 0)

        for b in range(NBUF):
            s = s_len - NBUF + b
            wait_gather(b)
            wait_write(b)
            transpose(b)
            fire_write(s, b)
        for b in range(NBUF):
            wait_write(b)

    return gather_tr


@jax.jit
def kernel(tokens, table):
    b, s = tokens.shape
    assert b % BW == 0 and (b // BW) == NW and s % NBUF == 0
    mesh = plsc.VectorSubcoreMesh(core_axis_name="c", subcore_axis_name="s")
    tail = jnp.concatenate(
        [table[VOCAB - BW // 2 :].T, jnp.zeros((EMB_DIM, BW // 2), jnp.float32)],
        axis=1,
    )
    tab2 = _make_transpose(mesh)(table.T, tail)
    tab_rm = tab2.reshape(VOCAB, EMB_DIM)
    tok_t = tokens.T.astype(jnp.int32)
    out5 = _make_gather(s, b // BW, mesh)(tab_rm, tok_t)
    return out5.transpose(2, 4, 0, 1, 3).reshape(b, s, EMB_DIM)


# speedup vs baseline: 1.3649x; 1.2669x over previous
import functools
import math

import jax
import jax.numpy as jnp
from jax import lax
from jax.experimental import pallas as pl
from jax.experimental.pallas import tpu as pltpu
from jax.experimental.pallas import tpu_sc as plsc

EMB_DIM = 64
SCALE = math.sqrt(EMB_DIM)

NC = 2
NS = 16
NW = NC * NS
LANES = 16

BW = 128
GRP = 8
NBUF = 4
PITCH = BW + 1


def _make_kernel(s_len, n_win):
    mesh = plsc.VectorSubcoreMesh(core_axis_name="c", subcore_axis_name="s")
    n_grp = EMB_DIM // GRP

    @functools.partial(
        pl.kernel,
        out_type=jax.ShapeDtypeStruct((s_len, n_grp, n_win, GRP, BW), jnp.float32),
        mesh=mesh,
        scratch_types=[
            pltpu.VMEM((s_len, BW), jnp.int32),
            [pltpu.VMEM((BW, EMB_DIM), jnp.float32) for _ in range(NBUF)],
            [pltpu.VMEM((GRP, GRP, PITCH), jnp.float32) for _ in range(NBUF)],
            [pltpu.SemaphoreType.DMA for _ in range(NBUF)],
            [pltpu.SemaphoreType.DMA for _ in range(NBUF)],
        ],
        compiler_params=pltpu.CompilerParams(
            use_tc_tiling_on_sc=False, needs_layout_passes=False
        ),
    )
    def gather_tr(table_hbm, tok_hbm, out_hbm, idx_v, stg, obuf, sg, sw):
        w = lax.axis_index("s") * NC + lax.axis_index("c")
        pltpu.sync_copy(tok_hbm.at[:, pl.ds(w * BW, BW)], idx_v)

        d_iota = lax.iota(jnp.int32, LANES)

        def fire_gather(s, b):
            pltpu.async_copy(table_hbm.at[idx_v.at[s]], stg[b], sg[b])

        def wait_gather(b):
            pltpu.make_async_copy(table_hbm.at[idx_v.at[0]], stg[b], sg[b]).wait()

        def fire_write(s, b):
            pltpu.async_copy(
                obuf[b].at[:, :, pl.ds(0, BW)], out_hbm.at[s, :, w], sw[b]
            )

        def wait_write(b):
            pltpu.make_async_copy(
                obuf[b].at[:, :, pl.ds(0, BW)], out_hbm.at[0, :, w], sw[b]
            ).wait()

        e_bases = [d_iota + (ev * LANES) for ev in range(EMB_DIM // LANES)]
        g_bases = [eb // GRP for eb in e_bases]
        r_bases = [eb % GRP for eb in e_bases]

        def transpose(b):
            @plsc.parallel_loop(0, BW, unroll=4)
            def d_body(d):
                d_splat = jnp.full((LANES,), 0, jnp.int32) + d
                for ev in range(EMB_DIM // LANES):
                    vals = stg[b][d, pl.ds(ev * LANES, LANES)] * SCALE
                    plsc.store_scatter(
                        obuf[b], [g_bases[ev], r_bases[ev], d_splat], vals
                    )

        for b in range(NBUF):
            fire_gather(b, b)
        for b in range(NBUF):
            wait_gather(b)
            transpose(b)
            fire_write(b, b)
            fire_gather(b + NBUF, b)

        def main_body(jj, _):
            for b in range(NBUF):
                s = jj * NBUF + b
                wait_gather(b)
                wait_write(b)
                transpose(b)
                fire_write(s, b)
                fire_gather(s + NBUF, b)
            return 0

        lax.fori_loop(1, s_len // NBUF - 1, main_body, 0)

        for b in range(NBUF):
            s = s_len - NBUF + b
            wait_gather(b)
            wait_write(b)
            transpose(b)
            fire_write(s, b)
        for b in range(NBUF):
            wait_write(b)

    return gather_tr


@jax.jit
def kernel(tokens, table):
    b, s = tokens.shape
    assert b % BW == 0 and (b // BW) == NW and s % NBUF == 0
    tok_t = tokens.T.astype(jnp.int32)
    out5 = _make_kernel(s, b // BW)(table, tok_t)
    return out5.transpose(2, 4, 0, 1, 3).reshape(b, s, EMB_DIM)
